# Initial kernel scaffold; baseline (speedup 1.0000x reference)
import jax
import jax.numpy as jnp
from jax.experimental import pallas as pl


def kernel(v_types, v_pos, adj, feats, W_ih, W_hh, b_ih, b_hh, gate_W, gate_b, map_W, df_W1, df_b1, df_W2, df_b2, fc1_W, fc1_b, fc2_W, fc2_b):
    raise NotImplementedError("write your pallas kernel here")


# speedup vs baseline: 3.2078x; 3.2078x over previous
import functools

import jax
import jax.numpy as jnp
from jax.experimental import pallas as pl

MAX_N = 8
NVT = 26
MAX_POS = 9
HS = 301
HSP = 384
EMB = 16
FEAT_EMB = 8
NZ = 56
XDIM = NVT + MAX_POS
B = 4096
BB = 512


def _sig(x):
    return jax.nn.sigmoid(x)


def _dot(a, b):
    return jax.lax.dot_general(a, b, (((1,), (0,)), ((), ())),
                               preferred_element_type=jnp.float32)


def _gru(gi, gh, hin):
    r = _sig(gi[:, 0:HSP] + gh[:, 0:HSP])
    z = _sig(gi[:, HSP:2 * HSP] + gh[:, HSP:2 * HSP])
    n = jnp.tanh(gi[:, 2 * HSP:3 * HSP] + r * gh[:, 2 * HSP:3 * HSP])
    return (1.0 - z) * n + z * hin


def _kernel_body(vt_ref, vp_ref, adj_ref, feats_ref,
                 wih_ref, whh_ref, bih_ref, bhh_ref,
                 wgm_ref, wgmp_ref, bgm_ref,
                 dfw1_ref, dfb1_ref, dfw2_ref, dfb2_ref,
                 fc1h_ref, fc1d_ref, fc1b_ref,
                 fc2h_ref, fc2d_ref, fc2b_ref,
                 mu_ref, lv_ref):
    vt = vt_ref[...]
    vp = vp_ref[...]
    bih = bih_ref[...]
    bhh = bhh_ref[...]
    bgm = bgm_ref[...]

    iota_x = jax.lax.broadcasted_iota(jnp.int32, (BB, XDIM), 1)
    iota_p = jax.lax.broadcasted_iota(jnp.int32, (BB, MAX_POS), 1)

    def onehot_x(v):
        t = vt[:, v:v + 1]
        p = vp[:, v:v + 1] + NVT
        return ((iota_x == t) | (iota_x == p)).astype(jnp.float32)

    def onehot_p(v):
        return (iota_p == vp[:, v:v + 1]).astype(jnp.float32)

    def gi_of(v):
        return _dot(onehot_x(v), wih_ref[...]) + bih

    h = _gru(gi_of(0), jnp.broadcast_to(bhh, (BB, 3 * HSP)), jnp.zeros((BB, HSP), jnp.float32))

    gated = []
    for v in range(1, MAX_N):
        u = v - 1
        gm = _dot(h, wgm_ref[...]) + _dot(onehot_p(u), wgmp_ref[...]) + bgm
        gated.append(_sig(gm[:, 0:HSP]) * gm[:, HSP:2 * HSP])
        hagg = jnp.zeros((BB, HSP), jnp.float32)
        for u2 in range(v):
            m = adj_ref[:, u2 * MAX_N + v:u2 * MAX_N + v + 1].astype(jnp.float32)
            hagg = hagg + m * gated[u2]
        gh = _dot(hagg, whh_ref[...]) + bhh
        h = _gru(gi_of(v), gh, hagg)

    iota_d = jax.lax.broadcasted_iota(jnp.int32, (BB, 3 * MAX_POS), 1) // 3
    df = jnp.zeros((BB, 3 * MAX_POS), jnp.float32)
    for v in range(MAX_N):
        fv = feats_ref[:, 3 * v:3 * v + 3]
        newv = jnp.concatenate([fv] * MAX_POS, axis=1)
        df = jnp.where(iota_d == vp[:, v:v + 1], newv, df)

    hd1 = jnp.maximum(_dot(df, dfw1_ref[...]) + dfb1_ref[...], 0.0)
    hd = _dot(hd1, dfw2_ref[...]) + dfb2_ref[...]

    mu_ref[...] = _dot(h, fc1h_ref[...]) + _dot(hd, fc1d_ref[...]) + fc1b_ref[...]
    lv_ref[...] = _dot(h, fc2h_ref[...]) + _dot(hd, fc2d_ref[...]) + fc2b_ref[...]


def _pad_rows(w, rows):
    return jnp.pad(w, ((0, rows - w.shape[0]), (0, 0)))


def _pad3(wT):
    k = wT.shape[0]
    out = jnp.zeros((k, 3 * HSP), wT.dtype)
    for c in range(3):
        out = out.at[:, c * HSP:c * HSP + HS].set(wT[:, c * HS:(c + 1) * HS])
    return out


@jax.jit
def kernel(v_types, v_pos, adj, feats, W_ih, W_hh, b_ih, b_hh, gate_W, gate_b,
           map_W, df_W1, df_b1, df_W2, df_b2, fc1_W, fc1_b, fc2_W, fc2_b):
    f32 = jnp.float32
    wih = _pad3(W_ih.T)
    whh = _pad_rows(_pad3(W_hh.T), HSP)
    bih = _pad3(b_ih[None, :])
    bhh = _pad3(b_hh[None, :])
    gW_h, gW_p = gate_W[:, :HS], gate_W[:, HS:]
    mW_h, mW_p = map_W[:, :HS], map_W[:, HS:]
    pad_h = lambda w: _pad_rows(jnp.pad(w.T, ((0, 0), (0, HSP - HS))), HSP)
    wgm = jnp.concatenate([pad_h(gW_h), pad_h(mW_h)], axis=1)
    pad_p = lambda w: jnp.pad(w.T, ((0, 0), (0, HSP - HS)))
    wgmp = jnp.concatenate([pad_p(gW_p), pad_p(mW_p)], axis=1)
    bgm = jnp.pad(gate_b[None, :], ((0, 0), (0, 2 * HSP - HS)))

    dfw1 = df_W1.T
    dfb1 = df_b1[None, :]
    dfw2 = df_W2.T
    dfb2 = df_b2[None, :]
    fc1h = _pad_rows(fc1_W[:, :HS].T, HSP)
    fc1d = fc1_W[:, HS:].T
    fc1b = fc1_b[None, :]
    fc2h = _pad_rows(fc2_W[:, :HS].T, HSP)
    fc2d = fc2_W[:, HS:].T
    fc2b = fc2_b[None, :]

    adj2 = adj.reshape(B, MAX_N * MAX_N)
    feats2 = feats.reshape(B, MAX_N * 3)

    nb = B // BB
    data_spec = lambda cols: pl.BlockSpec((BB, cols), lambda i: (i, 0))
    w_spec = lambda r, c: pl.BlockSpec((r, c), lambda i: (0, 0))

    in_specs = [
        data_spec(MAX_N), data_spec(MAX_N), data_spec(MAX_N * MAX_N),
        data_spec(MAX_N * 3),
        w_spec(XDIM, 3 * HSP), w_spec(HSP, 3 * HSP),
        w_spec(1, 3 * HSP), w_spec(1, 3 * HSP),
        w_spec(HSP, 2 * HSP), w_spec(MAX_POS, 2 * HSP), w_spec(1, 2 * HSP),
        w_spec(3 * MAX_POS, EMB), w_spec(1, EMB),
        w_spec(EMB, FEAT_EMB), w_spec(1, FEAT_EMB),
        w_spec(HSP, NZ), w_spec(FEAT_EMB, NZ), w_spec(1, NZ),
        w_spec(HSP, NZ), w_spec(FEAT_EMB, NZ), w_spec(1, NZ),
    ]
    out_specs = [data_spec(NZ), data_spec(NZ)]
    out_shape = [jax.ShapeDtypeStruct((B, NZ), f32)] * 2

    mu, lv = pl.pallas_call(
        _kernel_body,
        grid=(nb,),
        in_specs=in_specs,
        out_specs=out_specs,
        out_shape=out_shape,
    )(v_types.astype(jnp.int32), v_pos.astype(jnp.int32),
      adj2.astype(jnp.int32), feats2.astype(f32),
      wih, whh, bih, bhh, wgm, wgmp, bgm,
      dfw1, dfb1, dfw2, dfb2, fc1h, fc1d, fc1b, fc2h, fc2d, fc2b)
    return mu, lv


# speedup vs baseline: 3.3210x; 1.0353x over previous
import functools

import jax
import jax.numpy as jnp
from jax.experimental import pallas as pl

MAX_N = 8
NVT = 26
MAX_POS = 9
HS = 301
HSP = 384
EMB = 16
FEAT_EMB = 8
NZ = 56
XDIM = NVT + MAX_POS
B = 4096
BB = 512


def _sig(x):
    return jax.nn.sigmoid(x)


def _dot(a, b):
    return jax.lax.dot_general(a.astype(jnp.bfloat16), b, (((1,), (0,)), ((), ())),
                               preferred_element_type=jnp.float32)


def _gru(gi, gh, hin):
    r = _sig(gi[:, 0:HSP] + gh[:, 0:HSP])
    z = _sig(gi[:, HSP:2 * HSP] + gh[:, HSP:2 * HSP])
    n = jnp.tanh(gi[:, 2 * HSP:3 * HSP] + r * gh[:, 2 * HSP:3 * HSP])
    return (1.0 - z) * n + z * hin


def _kernel_body(vt_ref, vp_ref, adj_ref, feats_ref,
                 wih_ref, whh_ref, bih_ref, bhh_ref,
                 wgm_ref, wgmp_ref, bgm_ref,
                 dfw1_ref, dfb1_ref, dfw2_ref, dfb2_ref,
                 fc1h_ref, fc1d_ref, fc1b_ref,
                 fc2h_ref, fc2d_ref, fc2b_ref,
                 mu_ref, lv_ref):
    vt = vt_ref[...]
    vp = vp_ref[...]
    bih = bih_ref[...]
    bhh = bhh_ref[...]
    bgm = bgm_ref[...]

    iota_x = jax.lax.broadcasted_iota(jnp.int32, (BB, XDIM), 1)
    iota_p = jax.lax.broadcasted_iota(jnp.int32, (BB, MAX_POS), 1)

    def onehot_x(v):
        t = vt[:, v:v + 1]
        p = vp[:, v:v + 1] + NVT
        return ((iota_x == t) | (iota_x == p)).astype(jnp.float32)

    def onehot_p(v):
        return (iota_p == vp[:, v:v + 1]).astype(jnp.float32)

    def gi_of(v):
        return _dot(onehot_x(v), wih_ref[...]) + bih

    h = _gru(gi_of(0), jnp.broadcast_to(bhh, (BB, 3 * HSP)), jnp.zeros((BB, HSP), jnp.float32))

    gated = []
    for v in range(1, MAX_N):
        u = v - 1
        gm = _dot(h, wgm_ref[...]) + _dot(onehot_p(u), wgmp_ref[...]) + bgm
        gated.append(_sig(gm[:, 0:HSP]) * gm[:, HSP:2 * HSP])
        hagg = jnp.zeros((BB, HSP), jnp.float32)
        for u2 in range(v):
            m = adj_ref[:, u2 * MAX_N + v:u2 * MAX_N + v + 1].astype(jnp.float32)
            hagg = hagg + m * gated[u2]
        gh = _dot(hagg, whh_ref[...]) + bhh
        h = _gru(gi_of(v), gh, hagg)

    iota_d = jax.lax.broadcasted_iota(jnp.int32, (BB, 3 * MAX_POS), 1) // 3
    df = jnp.zeros((BB, 3 * MAX_POS), jnp.float32)
    for v in range(MAX_N):
        fv = feats_ref[:, 3 * v:3 * v + 3]
        newv = jnp.concatenate([fv] * MAX_POS, axis=1)
        df = jnp.where(iota_d == vp[:, v:v + 1], newv, df)

    hd1 = jnp.maximum(_dot(df, dfw1_ref[...]) + dfb1_ref[...], 0.0)
    hd = _dot(hd1, dfw2_ref[...]) + dfb2_ref[...]

    mu_ref[...] = _dot(h, fc1h_ref[...]) + _dot(hd, fc1d_ref[...]) + fc1b_ref[...]
    lv_ref[...] = _dot(h, fc2h_ref[...]) + _dot(hd, fc2d_ref[...]) + fc2b_ref[...]


def _pad_rows(w, rows):
    return jnp.pad(w, ((0, rows - w.shape[0]), (0, 0)))


def _pad3(wT):
    k = wT.shape[0]
    out = jnp.zeros((k, 3 * HSP), wT.dtype)
    for c in range(3):
        out = out.at[:, c * HSP:c * HSP + HS].set(wT[:, c * HS:(c + 1) * HS])
    return out


@jax.jit
def kernel(v_types, v_pos, adj, feats, W_ih, W_hh, b_ih, b_hh, gate_W, gate_b,
           map_W, df_W1, df_b1, df_W2, df_b2, fc1_W, fc1_b, fc2_W, fc2_b):
    f32 = jnp.float32
    wih = _pad3(W_ih.T)
    whh = _pad_rows(_pad3(W_hh.T), HSP)
    bih = _pad3(b_ih[None, :])
    bhh = _pad3(b_hh[None, :])
    gW_h, gW_p = gate_W[:, :HS], gate_W[:, HS:]
    mW_h, mW_p = map_W[:, :HS], map_W[:, HS:]
    pad_h = lambda w: _pad_rows(jnp.pad(w.T, ((0, 0), (0, HSP - HS))), HSP)
    wgm = jnp.concatenate([pad_h(gW_h), pad_h(mW_h)], axis=1)
    pad_p = lambda w: jnp.pad(w.T, ((0, 0), (0, HSP - HS)))
    wgmp = jnp.concatenate([pad_p(gW_p), pad_p(mW_p)], axis=1)
    bgm = jnp.pad(gate_b[None, :], ((0, 0), (0, 2 * HSP - HS)))

    dfw1 = df_W1.T
    dfb1 = df_b1[None, :]
    dfw2 = df_W2.T
    dfb2 = df_b2[None, :]
    fc1h = _pad_rows(fc1_W[:, :HS].T, HSP)
    fc1d = fc1_W[:, HS:].T
    fc1b = fc1_b[None, :]
    fc2h = _pad_rows(fc2_W[:, :HS].T, HSP)
    fc2d = fc2_W[:, HS:].T
    fc2b = fc2_b[None, :]

    adj2 = adj.reshape(B, MAX_N * MAX_N)
    feats2 = feats.reshape(B, MAX_N * 3)

    nb = B // BB
    data_spec = lambda cols: pl.BlockSpec((BB, cols), lambda i: (i, 0))
    w_spec = lambda r, c: pl.BlockSpec((r, c), lambda i: (0, 0))

    in_specs = [
        data_spec(MAX_N), data_spec(MAX_N), data_spec(MAX_N * MAX_N),
        data_spec(MAX_N * 3),
        w_spec(XDIM, 3 * HSP), w_spec(HSP, 3 * HSP),
        w_spec(1, 3 * HSP), w_spec(1, 3 * HSP),
        w_spec(HSP, 2 * HSP), w_spec(MAX_POS, 2 * HSP), w_spec(1, 2 * HSP),
        w_spec(3 * MAX_POS, EMB), w_spec(1, EMB),
        w_spec(EMB, FEAT_EMB), w_spec(1, FEAT_EMB),
        w_spec(HSP, NZ), w_spec(FEAT_EMB, NZ), w_spec(1, NZ),
        w_spec(HSP, NZ), w_spec(FEAT_EMB, NZ), w_spec(1, NZ),
    ]
    out_specs = [data_spec(NZ), data_spec(NZ)]
    out_shape = [jax.ShapeDtypeStruct((B, NZ), f32)] * 2

    mu, lv = pl.pallas_call(
        _kernel_body,
        grid=(nb,),
        in_specs=in_specs,
        out_specs=out_specs,
        out_shape=out_shape,
    )(v_types.astype(jnp.int32), v_pos.astype(jnp.int32),
      adj2.astype(jnp.int32), feats2.astype(f32),
      wih.astype(jnp.bfloat16), whh.astype(jnp.bfloat16), bih, bhh,
      wgm.astype(jnp.bfloat16), wgmp.astype(jnp.bfloat16), bgm,
      dfw1.astype(jnp.bfloat16), dfb1, dfw2.astype(jnp.bfloat16), dfb2,
      fc1h.astype(jnp.bfloat16), fc1d.astype(jnp.bfloat16), fc1b,
      fc2h.astype(jnp.bfloat16), fc2d.astype(jnp.bfloat16), fc2b)
    return mu, lv
